# Initial kernel scaffold; baseline (speedup 1.0000x reference)
import jax
import jax.numpy as jnp
from jax.experimental import pallas as pl


def kernel(input_ids, weight):
    raise NotImplementedError("write your pallas kernel here")


# speedup vs baseline: 3.5189x; 3.5189x over previous
import functools

import jax
import jax.numpy as jnp
from jax import lax
from jax.experimental import pallas as pl
from jax.experimental.pallas import tpu as pltpu
from jax.experimental.pallas import tpu_sc as plsc

NUM_EMB = 100000
DIM = 64
SCALE = 8.0

NC = 2
NS = 16
NW = NC * NS

CHUNK = 128


def _scale_body(w_ref, o_ref):
    o_ref[...] = w_ref[...] * SCALE


def _scale_table(weight):
    rows_per_blk = 2000
    return pl.pallas_call(
        _scale_body,
        out_shape=jax.ShapeDtypeStruct(weight.shape, weight.dtype),
        grid=(weight.shape[0] // rows_per_blk,),
        in_specs=[pl.BlockSpec((rows_per_blk, DIM), lambda i: (i, 0))],
        out_specs=pl.BlockSpec((rows_per_blk, DIM), lambda i: (i, 0)),
    )(weight)


def _make_gather(total_rows):
    assert total_rows % (NW * CHUNK * 2) == 0
    rows_per_w = total_rows // NW
    steps = rows_per_w // CHUNK
    mesh = plsc.VectorSubcoreMesh(
        core_axis_name="c", subcore_axis_name="s", num_cores=NC, num_subcores=NS
    )

    @functools.partial(
        pl.kernel,
        out_type=jax.ShapeDtypeStruct((total_rows, DIM), jnp.float32),
        mesh=mesh,
        scratch_types=[
            pltpu.VMEM((2, CHUNK), jnp.int32),
            pltpu.VMEM((2, CHUNK, DIM), jnp.float32),
            pltpu.SemaphoreType.DMA((2,)),
        ],
        compiler_params=pltpu.CompilerParams(use_tc_tiling_on_sc=False),
    )
    def gather(table_hbm, idx_hbm, out_hbm, idx_v, rows_v, sems):
        wid = lax.axis_index("s") * NC + lax.axis_index("c")
        base = wid * rows_per_w

        def fire(i, slot):
            off = base + i * CHUNK
            pltpu.sync_copy(idx_hbm.at[pl.ds(off, CHUNK)], idx_v.at[slot])
            pltpu.async_copy(
                table_hbm.at[idx_v.at[slot]], rows_v.at[slot], sems.at[slot]
            )

        def drain(i, slot):
            pltpu.make_async_copy(
                table_hbm.at[idx_v.at[slot]], rows_v.at[slot], sems.at[slot]
            ).wait()
            off = base + i * CHUNK
            pltpu.sync_copy(rows_v.at[slot], out_hbm.at[pl.ds(off, CHUNK)])

        fire(0, 0)

        @pl.loop(0, steps, step=2)
        def _(g):
            fire(g + 1, 1)
            drain(g, 0)

            @pl.when(g + 2 < steps)
            def _():
                fire(g + 2, 0)

            drain(g + 1, 1)

    return gather


def kernel(input_ids, weight):
    b, t = input_ids.shape
    total = b * t
    scaled = _scale_table(weight)
    flat_idx = input_ids.reshape(total)
    out = _make_gather(total)(scaled, flat_idx)
    return out.reshape(b, t, DIM)


# speedup vs baseline: 3.8630x; 1.0978x over previous
import functools

import jax
import jax.numpy as jnp
from jax import lax
from jax.experimental import pallas as pl
from jax.experimental.pallas import tpu as pltpu
from jax.experimental.pallas import tpu_sc as plsc

NUM_EMB = 100000
DIM = 64
SCALE = 8.0

NC = 2
NS = 16
NW = NC * NS

CHUNK = 128


def _scale_body(w_ref, o_ref):
    o_ref[...] = w_ref[...] * SCALE


def _scale_table(weight):
    rows_per_blk = 2000
    return pl.pallas_call(
        _scale_body,
        out_shape=jax.ShapeDtypeStruct(weight.shape, weight.dtype),
        grid=(weight.shape[0] // rows_per_blk,),
        in_specs=[pl.BlockSpec((rows_per_blk, DIM), lambda i: (i, 0))],
        out_specs=pl.BlockSpec((rows_per_blk, DIM), lambda i: (i, 0)),
    )(weight)


NBUF = 5


def _make_gather(total_rows):
    assert total_rows % (NW * CHUNK * NBUF) == 0
    rows_per_w = total_rows // NW
    steps = rows_per_w // CHUNK
    mesh = plsc.VectorSubcoreMesh(
        core_axis_name="c", subcore_axis_name="s", num_cores=NC, num_subcores=NS
    )

    @functools.partial(
        pl.kernel,
        out_type=jax.ShapeDtypeStruct((total_rows, DIM), jnp.float32),
        mesh=mesh,
        scratch_types=[
            pltpu.VMEM((rows_per_w,), jnp.int32),
            pltpu.VMEM((NBUF, CHUNK, DIM), jnp.float32),
            pltpu.SemaphoreType.DMA((NBUF,)),
        ],
        compiler_params=pltpu.CompilerParams(use_tc_tiling_on_sc=False),
    )
    def gather(table_hbm, idx_hbm, out_hbm, idx_v, rows_v, sems):
        wid = lax.axis_index("s") * NC + lax.axis_index("c")
        base = wid * rows_per_w

        pltpu.sync_copy(idx_hbm.at[pl.ds(base, rows_per_w)], idx_v)

        def fire(i, slot):
            off = pl.multiple_of(i * CHUNK, CHUNK)
            pltpu.async_copy(
                table_hbm.at[idx_v.at[pl.ds(off, CHUNK)]],
                rows_v.at[slot],
                sems.at[slot],
            )

        def drain(i, slot):
            off = pl.multiple_of(i * CHUNK, CHUNK)
            pltpu.make_async_copy(
                table_hbm.at[idx_v.at[pl.ds(off, CHUNK)]],
                rows_v.at[slot],
                sems.at[slot],
            ).wait()
            pltpu.sync_copy(
                rows_v.at[slot], out_hbm.at[pl.ds(base + i * CHUNK, CHUNK)]
            )

        for b in range(NBUF):
            fire(b, b)

        @pl.loop(0, steps, step=NBUF)
        def _(g):
            for b in range(NBUF):
                drain(g + b, b)

                @pl.when(g + b + NBUF < steps)
                def _():
                    fire(g + b + NBUF, b)

    return gather


def kernel(input_ids, weight):
    b, t = input_ids.shape
    total = b * t
    scaled = _scale_table(weight)
    flat_idx = input_ids.reshape(total)
    out = _make_gather(total)(scaled, flat_idx)
    return out.reshape(b, t, DIM)
